# baseline (speedup 1.0000x reference)
import functools

import jax
import jax.numpy as jnp
from jax import lax
from jax.experimental import pallas as pl
from jax.experimental.pallas import tpu as pltpu
from jax.experimental.pallas import tpu_sc as plsc

B = 16384
L = 20
NEG = 20
D = 64

NC, NS = 2, 16
NW = NC * NS
CH = 128
KG = 4


def _sc_gather(table_a, idx_a, table_b, idx_b):
    na, nb = idx_a.shape[0], idx_b.shape[0]
    ga = na // (NW * KG)
    gb = nb // (NW * KG)
    assert ga * NW * KG == na and gb * NW * KG == nb

    mesh = plsc.VectorSubcoreMesh(
        core_axis_name="c", subcore_axis_name="s",
        num_cores=NC, num_subcores=NS)

    @functools.partial(
        pl.kernel,
        out_type=(jax.ShapeDtypeStruct((na, CH, D), jnp.float32),
                  jax.ShapeDtypeStruct((nb, CH, D), jnp.float32)),
        mesh=mesh,
        scratch_types=[
            pltpu.VMEM((KG, CH), jnp.int32),
            pltpu.VMEM((KG, CH, D), jnp.float32),
            pltpu.SemaphoreType.DMA,
        ],
        compiler_params=pltpu.CompilerParams(use_tc_tiling_on_sc=False),
    )
    def k(ta, ia, tb, ib, oa, ob, idx_v, rows_v, sem):
        wid = lax.axis_index("s") * NC + lax.axis_index("c")

        def do_stream(tbl, ihbm, ohbm, groups):
            base = wid * groups * KG

            def body(g, carry):
                c0 = base + g * KG
                pltpu.sync_copy(ihbm.at[pl.ds(c0, KG)], idx_v)
                cops = [pltpu.async_copy(tbl.at[idx_v.at[j]], rows_v.at[j], sem)
                        for j in range(KG)]
                for c in cops:
                    c.wait()
                pltpu.sync_copy(rows_v, ohbm.at[pl.ds(c0, KG)])
                return carry

            lax.fori_loop(0, groups, body, 0)

        do_stream(ta, ia, oa, ga)
        do_stream(tb, ib, ob, gb)

    return k(table_a, idx_a, table_b, idx_b)


TB = 512
NT = B // TB


def _logsig(x):
    return jnp.minimum(x, 0.0) - jnp.log1p(jnp.exp(-jnp.abs(x)))


def _tc_loss(center_rows, neg_rows, ctx_rows, cl, xl):
    def body(cen_ref, neg_ref, ctx_ref, cl_ref, xl_ref, out_ref, acc_ref):
        i = pl.program_id(0)
        m = lax.dot_general(cl_ref[...], xl_ref[...],
                            (((0,), (0,)), ((), ())),
                            preferred_element_type=jnp.float32)
        ctxm = lax.dot_general(ctx_ref[...], m,
                               (((1,), (1,)), ((), ())),
                               preferred_element_type=jnp.float32)
        ctx3 = ctxm.reshape(TB, L, D)
        cen = cen_ref[...]
        s_c = jnp.sum(cen[:, None, :] * ctx3, axis=-1)
        acc = jnp.sum(_logsig(s_c))
        neg = neg_ref[...].reshape(TB, NEG, D)
        for j in range(NEG):
            s_j = jnp.sum(neg[:, j, :][:, None, :] * ctx3, axis=-1)
            acc += jnp.sum(_logsig(-s_j))

        @pl.when(i == 0)
        def _():
            acc_ref[0, 0] = acc

        @pl.when(i > 0)
        def _():
            acc_ref[0, 0] += acc

        @pl.when(i == NT - 1)
        def _():
            out_ref[0, 0] = -acc_ref[0, 0] / float(B * (1 + NEG) * L)

    res = pl.pallas_call(
        body,
        grid=(NT,),
        in_specs=[
            pl.BlockSpec((TB, D), lambda i: (i, 0)),
            pl.BlockSpec((TB * NEG, D), lambda i: (i, 0)),
            pl.BlockSpec((TB * L, D), lambda i: (i, 0)),
            pl.BlockSpec((128, D), lambda i: (0, 0)),
            pl.BlockSpec((128, D), lambda i: (0, 0)),
        ],
        out_specs=pl.BlockSpec(memory_space=pltpu.SMEM),
        out_shape=jax.ShapeDtypeStruct((1, 1), jnp.float32),
        scratch_shapes=[pltpu.SMEM((1, 1), jnp.float32)],
    )(center_rows, neg_rows, ctx_rows, cl, xl)
    return res[0, 0]


def kernel(center_embedding, center_linear, context_embedding, context_linear,
           center, contexts, negatives):
    idx_a = jnp.concatenate(
        [center.astype(jnp.int32), negatives.astype(jnp.int32).reshape(-1)]
    ).reshape(-1, CH)
    idx_b = contexts.astype(jnp.int32).reshape(-1, CH)

    rows_a, rows_b = _sc_gather(center_embedding, idx_a,
                                context_embedding, idx_b)

    flat_a = rows_a.reshape(-1, D)
    center_rows = flat_a[:B]
    neg_rows = flat_a[B:]
    ctx_rows = rows_b.reshape(-1, D)

    return _tc_loss(center_rows, neg_rows, ctx_rows,
                    center_linear, context_linear)


# speedup vs baseline: 3.8996x; 3.8996x over previous
import functools

import jax
import jax.numpy as jnp
from jax import lax
from jax.experimental import pallas as pl
from jax.experimental.pallas import tpu as pltpu
from jax.experimental.pallas import tpu_sc as plsc

B = 16384
L = 20
NEG = 20
D = 64

NC, NS = 2, 16
NW = NC * NS
CH = 128
KG = 4


def _sc_gather(table_a, idx_a, table_b, idx_b):
    na, nb = idx_a.shape[0], idx_b.shape[0]
    ga = na // (NW * KG)
    gb = nb // (NW * KG)
    assert ga * NW * KG == na and gb * NW * KG == nb

    mesh = plsc.VectorSubcoreMesh(
        core_axis_name="c", subcore_axis_name="s",
        num_cores=NC, num_subcores=NS)

    @functools.partial(
        pl.kernel,
        out_type=(jax.ShapeDtypeStruct((na, CH, D), jnp.float32),
                  jax.ShapeDtypeStruct((nb, CH, D), jnp.float32)),
        mesh=mesh,
        scratch_types=[
            pltpu.VMEM((KG, CH), jnp.int32),
            pltpu.VMEM((KG, CH, D), jnp.float32),
            pltpu.SemaphoreType.DMA,
        ],
        compiler_params=pltpu.CompilerParams(use_tc_tiling_on_sc=False),
    )
    def k(ta, ia, tb, ib, oa, ob, idx_v, rows_v, sem):
        wid = lax.axis_index("s") * NC + lax.axis_index("c")

        def do_stream(tbl, ihbm, ohbm, groups):
            base = wid * groups * KG

            def body(g, carry):
                c0 = base + g * KG
                pltpu.sync_copy(ihbm.at[pl.ds(c0, KG)], idx_v)
                cops = [pltpu.async_copy(tbl.at[idx_v.at[j]], rows_v.at[j], sem)
                        for j in range(KG)]
                for c in cops:
                    c.wait()
                pltpu.sync_copy(rows_v, ohbm.at[pl.ds(c0, KG)])
                return carry

            lax.fori_loop(0, groups, body, 0)

        do_stream(ta, ia, oa, ga)
        do_stream(tb, ib, ob, gb)

    return k(table_a, idx_a, table_b, idx_b)


TB = 512
NT = B // TB


def _logsig(x):
    return jnp.minimum(x, 0.0) - jnp.log1p(jnp.exp(-jnp.abs(x)))


def _tc_loss(center_rows, neg_rows, ctx_rows, cl, xl):
    def body(cen_ref, neg_ref, ctx_ref, cl_ref, xl_ref, out_ref, acc_ref):
        i = pl.program_id(0)
        m = lax.dot_general(cl_ref[...], xl_ref[...],
                            (((0,), (0,)), ((), ())),
                            preferred_element_type=jnp.float32)
        cen = cen_ref[...]
        neg = neg_ref[...].reshape(NEG * TB, D)
        ctx = ctx_ref[...].reshape(L * TB, D)
        am_c = lax.dot_general(cen, m, (((1,), (0,)), ((), ())),
                               preferred_element_type=jnp.float32)
        am_n = lax.dot_general(neg, -m, (((1,), (0,)), ((), ())),
                               preferred_element_type=jnp.float32)
        act = am_c.T
        ant = am_n.T
        ct = ctx.T
        rows = []
        for n in range(1 + NEG):
            a_n = act if n == 0 else ant[:, (n - 1) * TB:n * TB]
            for l in range(L):
                c_l = ct[:, l * TB:(l + 1) * TB]
                rows.append(jnp.sum(a_n * c_l, axis=0))
        s_all = jnp.stack(rows)
        acc = jnp.sum(_logsig(s_all))

        @pl.when(i == 0)
        def _():
            acc_ref[0, 0] = acc

        @pl.when(i > 0)
        def _():
            acc_ref[0, 0] += acc

        @pl.when(i == NT - 1)
        def _():
            out_ref[0, 0] = -acc_ref[0, 0] / float(B * (1 + NEG) * L)

    res = pl.pallas_call(
        body,
        grid=(NT,),
        in_specs=[
            pl.BlockSpec((TB, D), lambda i: (i, 0)),
            pl.BlockSpec((NEG, TB, D), lambda i: (0, i, 0)),
            pl.BlockSpec((L, TB, D), lambda i: (0, i, 0)),
            pl.BlockSpec((128, D), lambda i: (0, 0)),
            pl.BlockSpec((128, D), lambda i: (0, 0)),
        ],
        out_specs=pl.BlockSpec(memory_space=pltpu.SMEM),
        out_shape=jax.ShapeDtypeStruct((1, 1), jnp.float32),
        scratch_shapes=[pltpu.SMEM((1, 1), jnp.float32)],
    )(center_rows, neg_rows, ctx_rows, cl, xl)
    return res[0, 0]


def kernel(center_embedding, center_linear, context_embedding, context_linear,
           center, contexts, negatives):
    idx_a = jnp.concatenate(
        [center.astype(jnp.int32), negatives.astype(jnp.int32).T.reshape(-1)]
    ).reshape(-1, CH)
    idx_b = contexts.astype(jnp.int32).T.reshape(-1, CH)

    rows_a, rows_b = _sc_gather(center_embedding, idx_a,
                                context_embedding, idx_b)

    flat_a = rows_a.reshape(-1, D)
    center_rows = flat_a[:B]
    neg_rows = flat_a[B:].reshape(NEG, B, D)
    ctx_rows = rows_b.reshape(L, B, D)

    return _tc_loss(center_rows, neg_rows, ctx_rows,
                    center_linear, context_linear)


# speedup vs baseline: 5.0818x; 1.3032x over previous
import functools

import jax
import jax.numpy as jnp
from jax import lax
from jax.experimental import pallas as pl
from jax.experimental.pallas import tpu as pltpu
from jax.experimental.pallas import tpu_sc as plsc

B = 16384
L = 20
NEG = 20
D = 64

NC, NS = 2, 16
NW = NC * NS
CH = 128
KG = 4


GR = KG * CH


def _sc_gather(table_a, idx_c, idx_n, table_b, idx_x):
    mesh = plsc.VectorSubcoreMesh(
        core_axis_name="c", subcore_axis_name="s",
        num_cores=NC, num_subcores=NS)

    @functools.partial(
        pl.kernel,
        out_type=(jax.ShapeDtypeStruct((B, D), jnp.float32),
                  jax.ShapeDtypeStruct((NEG, B, D), jnp.float32),
                  jax.ShapeDtypeStruct((L, B, D), jnp.float32)),
        mesh=mesh,
        scratch_types=[
            pltpu.VMEM((KG, CH), jnp.int32),
            pltpu.VMEM((GR, D), jnp.float32),
            pltpu.SemaphoreType.DMA,
        ],
        compiler_params=pltpu.CompilerParams(use_tc_tiling_on_sc=False),
    )
    def k(ta, ic, in_, tb, ix, oc, on, ox, idx_v, rows_v, sem):
        wid = lax.axis_index("s") * NC + lax.axis_index("c")

        def fetch(tbl, ihbm, c0):
            pltpu.sync_copy(ihbm.at[pl.ds(c0, KG)], idx_v)
            cops = [pltpu.async_copy(tbl.at[idx_v.at[j]],
                                     rows_v.at[pl.ds(j * CH, CH)], sem)
                    for j in range(KG)]
            for c in cops:
                c.wait()

        fetch(ta, ic, wid * KG)
        pltpu.sync_copy(rows_v, oc.at[pl.ds(wid * GR, GR)])

        def do_stream(tbl, ihbm, ohbm):
            def body(g, carry):
                r0 = wid * (NEG * B // NW) + g * GR
                n = r0 // B
                b0 = r0 - n * B
                fetch(tbl, ihbm, r0 // CH)
                pltpu.sync_copy(rows_v, ohbm.at[n, pl.ds(b0, GR)])
                return carry

            lax.fori_loop(0, NEG * B // NW // GR, body, 0)

        do_stream(ta, in_, on)
        do_stream(tb, ix, ox)

    return k(table_a, idx_c, idx_n, table_b, idx_x)


TB = 512
NT = B // TB


def _logsig(x):
    return jnp.minimum(x, 0.0) - jnp.log1p(jnp.exp(-jnp.abs(x)))


def _tc_loss(center_rows, neg_rows, ctx_rows, cl, xl):
    def body(cen_ref, neg_ref, ctx_ref, cl_ref, xl_ref, out_ref, acc_ref):
        i = pl.program_id(0)
        m = lax.dot_general(cl_ref[...], xl_ref[...],
                            (((0,), (0,)), ((), ())),
                            preferred_element_type=jnp.float32)
        cen = cen_ref[...]
        neg = neg_ref[...].reshape(NEG * TB, D)
        ctx = ctx_ref[...].reshape(L * TB, D)
        am_c = lax.dot_general(cen, m, (((1,), (0,)), ((), ())),
                               preferred_element_type=jnp.float32)
        am_n = lax.dot_general(neg, -m, (((1,), (0,)), ((), ())),
                               preferred_element_type=jnp.float32)
        act = am_c.T
        ant = am_n.T
        ct = ctx.T
        rows = []
        for n in range(1 + NEG):
            a_n = act if n == 0 else ant[:, (n - 1) * TB:n * TB]
            for l in range(L):
                c_l = ct[:, l * TB:(l + 1) * TB]
                rows.append(jnp.sum(a_n * c_l, axis=0))
        s_all = jnp.stack(rows)
        acc = jnp.sum(_logsig(s_all))

        @pl.when(i == 0)
        def _():
            acc_ref[0, 0] = acc

        @pl.when(i > 0)
        def _():
            acc_ref[0, 0] += acc

        @pl.when(i == NT - 1)
        def _():
            out_ref[0, 0] = -acc_ref[0, 0] / float(B * (1 + NEG) * L)

    res = pl.pallas_call(
        body,
        grid=(NT,),
        in_specs=[
            pl.BlockSpec((TB, D), lambda i: (i, 0)),
            pl.BlockSpec((NEG, TB, D), lambda i: (0, i, 0)),
            pl.BlockSpec((L, TB, D), lambda i: (0, i, 0)),
            pl.BlockSpec((128, D), lambda i: (0, 0)),
            pl.BlockSpec((128, D), lambda i: (0, 0)),
        ],
        out_specs=pl.BlockSpec(memory_space=pltpu.SMEM),
        out_shape=jax.ShapeDtypeStruct((1, 1), jnp.float32),
        scratch_shapes=[pltpu.SMEM((1, 1), jnp.float32)],
    )(center_rows, neg_rows, ctx_rows, cl, xl)
    return res[0, 0]


def kernel(center_embedding, center_linear, context_embedding, context_linear,
           center, contexts, negatives):
    idx_c = center.astype(jnp.int32).reshape(-1, CH)
    idx_n = negatives.astype(jnp.int32).T.reshape(-1, CH)
    idx_x = contexts.astype(jnp.int32).T.reshape(-1, CH)

    center_rows, neg_rows, ctx_rows = _sc_gather(
        center_embedding, idx_c, idx_n, context_embedding, idx_x)

    return _tc_loss(center_rows, neg_rows, ctx_rows,
                    center_linear, context_linear)


# speedup vs baseline: 7.7261x; 1.5203x over previous
import functools

import jax
import jax.numpy as jnp
from jax import lax
from jax.experimental import pallas as pl
from jax.experimental.pallas import tpu as pltpu
from jax.experimental.pallas import tpu_sc as plsc

B = 16384
L = 20
NEG = 20
D = 64

NC, NS = 2, 16
NW = NC * NS
CH = 128
KG = 4
GR = KG * CH


def _sc_gather(table_a, idx_c, neg_flat, table_b, ctx_flat):
    mesh = plsc.VectorSubcoreMesh(
        core_axis_name="c", subcore_axis_name="s",
        num_cores=NC, num_subcores=NS)

    @functools.partial(
        pl.kernel,
        out_type=(jax.ShapeDtypeStruct((B // 2, 2 * D), jnp.float32),
                  jax.ShapeDtypeStruct((NEG, B // 2, 2 * D), jnp.float32),
                  jax.ShapeDtypeStruct((L, B // 2, 2 * D), jnp.float32)),
        mesh=mesh,
        scratch_types=[
            pltpu.VMEM((2, KG // 2, CH), jnp.int32),
            pltpu.VMEM((2, GR // 2, D), jnp.float32),
            pltpu.VMEM((GR * NEG,), jnp.int32),
            pltpu.SemaphoreType.DMA,
        ],
        compiler_params=pltpu.CompilerParams(use_tc_tiling_on_sc=False,
                                             needs_layout_passes=False),
    )
    def k(ta, ic, inn, tb, ixx, oc, on, ox, idx_v, rows_v, tile_v, sem):
        wid = lax.axis_index("s") * NC + lax.axis_index("c")
        b0 = wid * GR
        h0 = wid * (GR // 2)
        lanes = lax.iota(jnp.int32, 16)

        def fill_idx(base2):
            for par in range(2):
                for q in range(GR // 2 // 16):
                    flat = base2 + (2 * (lanes + q * 16) + par) * NEG
                    v = plsc.load_gather(tile_v, [flat])
                    idx_v[par, q // 8, pl.ds((q % 8) * 16, 16)] = v

        def gather_group(tbl):
            cops = [pltpu.async_copy(tbl.at[idx_v.at[par, j]],
                                     rows_v.at[par, pl.ds(j * CH, CH)], sem)
                    for par in range(2) for j in range(KG // 2)]
            for c in cops:
                c.wait()

        def scatter(dst):
            for par in range(2):
                pltpu.sync_copy(rows_v.at[par],
                                dst.at[pl.ds(h0, GR // 2),
                                       pl.ds(par * D, D)])

        pltpu.sync_copy(ic.at[pl.ds(b0, GR)], tile_v.at[pl.ds(0, GR)])
        for par in range(2):
            for q in range(GR // 2 // 16):
                v = plsc.load_gather(tile_v, [2 * (lanes + q * 16) + par])
                idx_v[par, q // 8, pl.ds((q % 8) * 16, 16)] = v
        gather_group(ta)
        scatter(oc)

        def do_stream(tbl, ihbm, ohbm):
            pltpu.sync_copy(ihbm.at[pl.ds(b0 * NEG, GR * NEG)], tile_v)

            def body(n, carry):
                fill_idx(n)
                gather_group(tbl)
                scatter(ohbm.at[n])
                return carry

            lax.fori_loop(0, NEG, body, 0)

        do_stream(ta, inn, on)
        do_stream(tb, ixx, ox)

    return k(table_a, idx_c, neg_flat, table_b, ctx_flat)


TB = 512
TH = TB // 2
NT = B // TB


def _logsig(x):
    return jnp.minimum(x, 0.0) - jnp.log1p(jnp.exp(-jnp.abs(x)))


def _tc_loss(center_rows, neg_rows, ctx_rows, cl, xl):
    def body(cen_ref, neg_ref, ctx_ref, cl_ref, xl_ref, out_ref, acc_ref):
        i = pl.program_id(0)
        m = lax.dot_general(cl_ref[...], xl_ref[...],
                            (((0,), (0,)), ((), ())),
                            preferred_element_type=jnp.float32)
        z = jnp.zeros((D, D), jnp.float32)
        m2 = jnp.concatenate(
            [jnp.concatenate([m, z], axis=1),
             jnp.concatenate([z, m], axis=1)], axis=0)
        cen2 = cen_ref[...]
        neg2 = neg_ref[...].reshape(NEG * TH, 2 * D)
        ctx2 = ctx_ref[...].reshape(L * TH, 2 * D)
        amc = lax.dot_general(cen2, m2, (((1,), (0,)), ((), ())),
                              preferred_element_type=jnp.float32)
        amn = lax.dot_general(neg2, -m2, (((1,), (0,)), ((), ())),
                              preferred_element_type=jnp.float32)
        act = amc.T
        ant = amn.T
        ct = ctx2.T
        rows = []
        for n in range(1 + NEG):
            a_n = act if n == 0 else ant[:, (n - 1) * TH:n * TH]
            for l in range(L):
                p = a_n * ct[:, l * TH:(l + 1) * TH]
                rows.append(jnp.sum(p[:D], axis=0))
                rows.append(jnp.sum(p[D:], axis=0))
        s_all = jnp.stack(rows)
        acc = jnp.sum(_logsig(s_all))

        @pl.when(i == 0)
        def _():
            acc_ref[0, 0] = acc

        @pl.when(i > 0)
        def _():
            acc_ref[0, 0] += acc

        @pl.when(i == NT - 1)
        def _():
            out_ref[0, 0] = -acc_ref[0, 0] / float(B * (1 + NEG) * L)

    res = pl.pallas_call(
        body,
        grid=(NT,),
        in_specs=[
            pl.BlockSpec((TH, 2 * D), lambda i: (i, 0)),
            pl.BlockSpec((NEG, TH, 2 * D), lambda i: (0, i, 0)),
            pl.BlockSpec((L, TH, 2 * D), lambda i: (0, i, 0)),
            pl.BlockSpec((128, D), lambda i: (0, 0)),
            pl.BlockSpec((128, D), lambda i: (0, 0)),
        ],
        out_specs=pl.BlockSpec(memory_space=pltpu.SMEM),
        out_shape=jax.ShapeDtypeStruct((1, 1), jnp.float32),
        scratch_shapes=[pltpu.SMEM((1, 1), jnp.float32)],
    )(center_rows, neg_rows, ctx_rows, cl, xl)
    return res[0, 0]


def kernel(center_embedding, center_linear, context_embedding, context_linear,
           center, contexts, negatives):
    idx_c = center.astype(jnp.int32)
    neg_flat = negatives.astype(jnp.int32).reshape(-1)
    ctx_flat = contexts.astype(jnp.int32).reshape(-1)

    center_rows, neg_rows, ctx_rows = _sc_gather(
        center_embedding, idx_c, neg_flat, context_embedding, ctx_flat)

    return _tc_loss(center_rows, neg_rows, ctx_rows,
                    center_linear, context_linear)
